# baseline (speedup 1.0000x reference)
import functools

import jax
import jax.numpy as jnp
from jax import lax
from jax.experimental import pallas as pl
from jax.experimental.pallas import tpu as pltpu
from jax.experimental.pallas import tpu_sc as plsc

_B = 1024
_N = 4096
_DEG = 15
_EPS = 1.0
_K = (_N - 1) // 2
_NEG = -1.0e30

_NC = 2
_NS = 16
_NW = _NC * _NS
_ROWS_PER_W = _B // _NW
_SB = 16
_NSB = _ROWS_PER_W // _SB
_OC = 1024
_NOC = _N // _OC
_OG = _OC // 16

_IMIN = -2147483648


def _gate_body(x_ref, o_ref):
    x = x_ref[...]
    s = lax.bitcast_convert_type(x, jnp.int32)
    f = s ^ ((s >> 31) & jnp.int32(0x7FFFFFFF))
    rows = x.shape[0]
    p0 = jnp.zeros((rows, 1), jnp.int32)

    def bit_step(i, p):
        b = jnp.int32(1) << (31 - i)
        q = p | b
        t_f = q ^ jnp.int32(_IMIN)
        cnt = jnp.sum((f < t_f).astype(jnp.int32), axis=1, keepdims=True)
        return jnp.where(cnt <= _K, q, p)

    p = lax.fori_loop(0, 32, bit_step, p0)
    med_f = p ^ jnp.int32(_IMIN)
    med_s = med_f ^ ((med_f >> 31) & jnp.int32(0x7FFFFFFF))
    med = lax.bitcast_convert_type(med_s, jnp.float32)
    xt = jnp.where(jnp.abs(x - med) < _EPS, x, jnp.float32(_NEG))
    o_ref[...] = xt.reshape(-1)


def _gate_prep(x, rows_blk):
    rows = x.shape[0]
    return pl.pallas_call(
        _gate_body,
        grid=(rows // rows_blk,),
        in_specs=[pl.BlockSpec((rows_blk, _N), lambda i: (i, 0))],
        out_specs=pl.BlockSpec((rows_blk * _N,), lambda i: (i,)),
        out_shape=jax.ShapeDtypeStruct((rows * _N,), jnp.float32),
        compiler_params=pltpu.CompilerParams(
            dimension_semantics=("parallel",)),
    )(x)


def _sc_body(xt_hbm, idx_hbm, w_hbm, out_hbm, xrows, idxc, wc, outc):
    rows = xt_hbm.shape[0] // _N
    rows_per_w = rows // _NW
    sb_rows = xrows.shape[0] // _N
    nsb = rows_per_w // sb_rows
    cid = lax.axis_index("c")
    sid = lax.axis_index("s")
    wid = sid * _NC + cid
    rbase = wid * rows_per_w

    for sb in range(nsb):
        row0 = rbase + sb * sb_rows
        pltpu.sync_copy(xt_hbm.at[pl.ds(row0 * _N, sb_rows * _N)], xrows)
        for oc in range(_NOC):
            pltpu.sync_copy(idx_hbm.at[oc], idxc)
            pltpu.sync_copy(w_hbm.at[oc], wc)

            @plsc.parallel_loop(0, _OG)
            def og_body(og):
                idx_vs = [idxc[d, pl.ds(og * 16, 16)] for d in range(_DEG)]
                w_vs = [wc[d, pl.ds(og * 16, 16)] for d in range(_DEG)]

                @plsc.parallel_loop(0, sb_rows)
                def r_body(r):
                    xrow = xrows.at[pl.ds(r * _N, _N)]
                    vmax = None
                    vmin = None
                    for d in range(_DEG):
                        v = plsc.load_gather(xrow, [idx_vs[d]])
                        t = v + w_vs[d]
                        vmax = t if vmax is None else jnp.maximum(vmax, t)
                        vmin = t if vmin is None else jnp.minimum(vmin, t)
                    res = jnp.where(vmin < jnp.float32(-1e20),
                                    jnp.maximum(vmax, jnp.float32(0.0)), vmax)
                    outc[r, pl.ds(og * 16, 16)] = res
            pltpu.sync_copy(
                outc, out_hbm.at[pl.ds(row0, sb_rows), pl.ds(oc * _OC, _OC)])


def _sc_call(xt, idx_c, w_c):
    rows = xt.shape[0] // _N
    sb_rows = min(_SB, rows // _NW)
    mesh = plsc.VectorSubcoreMesh(core_axis_name="c", subcore_axis_name="s")
    return pl.kernel(
        _sc_body,
        out_type=jax.ShapeDtypeStruct((rows, _N), jnp.float32),
        mesh=mesh,
        scratch_types=[
            pltpu.VMEM((sb_rows * _N,), jnp.float32),
            pltpu.VMEM((_DEG, _OC), jnp.int32),
            pltpu.VMEM((_DEG, _OC), jnp.float32),
            pltpu.VMEM((sb_rows, _OC), jnp.float32),
        ],
        compiler_params=pltpu.CompilerParams(use_tc_tiling_on_sc=False,
                                             needs_layout_passes=False),
    )(xt, idx_c, w_c)


_NSLICE = 1


@functools.partial(jax.jit, static_argnums=())
def kernel(x, weights, synaptic_indices):
    src = synaptic_indices[1]
    idx_c = src.reshape(_N, _DEG).T.reshape(_DEG, _NOC, _OC).transpose(1, 0, 2)
    w_c = weights.reshape(_N, _DEG).T.reshape(_DEG, _NOC, _OC).transpose(1, 0, 2)
    idx_c = idx_c.astype(jnp.int32)
    rows = _B // _NSLICE
    outs = []
    for s in range(_NSLICE):
        xt = _gate_prep(lax.slice_in_dim(x, s * rows, (s + 1) * rows), 256)
        outs.append(_sc_call(xt, idx_c, w_c))
    return lax.concatenate(outs, 0)


# speedup vs baseline: 1.0157x; 1.0157x over previous
import functools

import jax
import jax.numpy as jnp
from jax import lax
from jax.experimental import pallas as pl
from jax.experimental.pallas import tpu as pltpu
from jax.experimental.pallas import tpu_sc as plsc

_B = 1024
_N = 4096
_DEG = 15
_EPS = 1.0
_K = (_N - 1) // 2
_NEG = -1.0e30

_NC = 2
_NS = 16
_NW = _NC * _NS
_ROWS_PER_W = _B // _NW
_SB = 16
_NSB = _ROWS_PER_W // _SB
_OC = 1024
_NOC = _N // _OC
_OG = _OC // 16

_IMIN = -2147483648


def _gate_body(x_ref, o_ref):
    x = x_ref[...]
    s = lax.bitcast_convert_type(x, jnp.int32)
    f = s ^ ((s >> 31) & jnp.int32(0x7FFFFFFF))
    rows = x.shape[0]
    p0 = jnp.zeros((rows, 1), jnp.int32)

    def bit_step(i, p):
        b = jnp.int32(1) << (31 - i)
        q = p | b
        t_f = q ^ jnp.int32(_IMIN)
        cnt = jnp.sum((f < t_f).astype(jnp.int32), axis=1, keepdims=True)
        return jnp.where(cnt <= _K, q, p)

    p = lax.fori_loop(0, 32, bit_step, p0)
    med_f = p ^ jnp.int32(_IMIN)
    med_s = med_f ^ ((med_f >> 31) & jnp.int32(0x7FFFFFFF))
    med = lax.bitcast_convert_type(med_s, jnp.float32)
    xt = jnp.where(jnp.abs(x - med) < _EPS, x, jnp.float32(_NEG))
    o_ref[...] = xt.reshape(-1)


def _gate_prep(x, rows_blk):
    rows = x.shape[0]
    return pl.pallas_call(
        _gate_body,
        grid=(rows // rows_blk,),
        in_specs=[pl.BlockSpec((rows_blk, _N), lambda i: (i, 0))],
        out_specs=pl.BlockSpec((rows_blk * _N,), lambda i: (i,)),
        out_shape=jax.ShapeDtypeStruct((rows * _N,), jnp.float32),
        compiler_params=pltpu.CompilerParams(
            dimension_semantics=("parallel",)),
    )(x)


def _sc_body(xt_hbm, idx_hbm, w_hbm, out_hbm, xrows, idxc, wc, outc):
    rows = xt_hbm.shape[0] // _N
    rows_per_w = rows // _NW
    sb_rows = xrows.shape[0] // _N
    nsb = rows_per_w // sb_rows
    cid = lax.axis_index("c")
    sid = lax.axis_index("s")
    wid = sid * _NC + cid
    rbase = wid * rows_per_w

    for sb in range(nsb):
        row0 = rbase + sb * sb_rows
        pltpu.sync_copy(xt_hbm.at[pl.ds(row0 * _N, sb_rows * _N)], xrows)
        for oc in range(_NOC):
            pltpu.sync_copy(idx_hbm.at[oc], idxc)
            pltpu.sync_copy(w_hbm.at[oc], wc)

            @plsc.parallel_loop(0, _OG)
            def og_body(og):
                idx_vs = [idxc[d, pl.ds(og * 16, 16)] for d in range(_DEG)]
                w_vs = [wc[d, pl.ds(og * 16, 16)] for d in range(_DEG)]

                @plsc.parallel_loop(0, sb_rows)
                def r_body(r):
                    xrow = xrows.at[pl.ds(r * _N, _N)]
                    vmax = None
                    vmin = None
                    for d in range(_DEG):
                        v = plsc.load_gather(xrow, [idx_vs[d]])
                        t = v + w_vs[d]
                        vmax = t if vmax is None else jnp.maximum(vmax, t)
                        vmin = t if vmin is None else jnp.minimum(vmin, t)
                    res = jnp.where(vmin < jnp.float32(-1e20),
                                    jnp.maximum(vmax, jnp.float32(0.0)), vmax)
                    outc[r, pl.ds(og * 16, 16)] = res
            pltpu.sync_copy(
                outc, out_hbm.at[pl.ds(row0, sb_rows), pl.ds(oc * _OC, _OC)])


def _sc_call(xt, idx_c, w_c):
    rows = xt.shape[0] // _N
    sb_rows = min(_SB, rows // _NW)
    mesh = plsc.VectorSubcoreMesh(core_axis_name="c", subcore_axis_name="s")
    return pl.kernel(
        _sc_body,
        out_type=jax.ShapeDtypeStruct((rows, _N), jnp.float32),
        mesh=mesh,
        scratch_types=[
            pltpu.VMEM((sb_rows * _N,), jnp.float32),
            pltpu.VMEM((_DEG, _OC), jnp.int32),
            pltpu.VMEM((_DEG, _OC), jnp.float32),
            pltpu.VMEM((sb_rows, _OC), jnp.float32),
        ],
        compiler_params=pltpu.CompilerParams(use_tc_tiling_on_sc=False,
                                             needs_layout_passes=False),
    )(xt, idx_c, w_c)


_NSLICE = 1


@functools.partial(jax.jit, static_argnums=())
def kernel(x, weights, synaptic_indices):
    src = synaptic_indices[1]
    idx_c = src.reshape(_N, _DEG).T.reshape(_DEG, _NOC, _OC).transpose(1, 0, 2)
    w_c = weights.reshape(_N, _DEG).T.reshape(_DEG, _NOC, _OC).transpose(1, 0, 2)
    idx_c = idx_c.astype(jnp.int32)
    rows = _B // _NSLICE
    outs = []
    for s in range(_NSLICE):
        xt = _gate_prep(lax.slice_in_dim(x, s * rows, (s + 1) * rows), 512)
        outs.append(_sc_call(xt, idx_c, w_c))
    return lax.concatenate(outs, 0)
